# baseline (speedup 1.0000x reference)
import functools

import jax
import jax.numpy as jnp
from jax import lax
from jax.experimental import pallas as pl
from jax.experimental.pallas import tpu as pltpu
from jax.experimental.pallas import tpu_sc as plsc

BATCH = 4096
HIST = 50
EMBED_DIM = 32

NC = 2
NS = 16
NW = NC * NS

BAGS_PER_W = BATCH // NW
BAGS_PER_STEP = 2
IDX_PER_STEP = BAGS_PER_STEP * HIST
STEPS = BAGS_PER_W // BAGS_PER_STEP
NBUF = 2

D2 = EMBED_DIM // 2


def _body(tbl_hbm, idx_hbm, out_hbm, idx_v, rows0, rows1, out_v, sem0, sem1):
    c = lax.axis_index("c")
    s = lax.axis_index("s")
    wid = s * NC + c
    ibase = wid * STEPS
    obase = wid * BAGS_PER_W

    pltpu.sync_copy(idx_hbm.at[pl.ds(ibase, STEPS)], idx_v)

    rows = (rows0, rows1)
    sems = (sem0, sem1)

    def gather(step, buf):
        pltpu.async_copy(tbl_hbm.at[idx_v.at[step]], rows[buf], sems[buf])

    gather(0, 0)
    gather(1, 1)

    def reduce_step(buf, step):
        rb = rows[buf]
        for r in range(BAGS_PER_STEP):
            off = r * HIST
            lo = rb[off, pl.ds(0, D2)]
            hi = rb[off, pl.ds(D2, D2)]
            for j in range(1, HIST):
                lo = lo + rb[off + j, pl.ds(0, D2)]
                hi = hi + rb[off + j, pl.ds(D2, D2)]
            orow = step * BAGS_PER_STEP + r
            out_v[orow, pl.ds(0, D2)] = lo
            out_v[orow, pl.ds(D2, D2)] = hi

    def outer(o, carry):
        for b in range(NBUF):
            step = o * NBUF + b
            pltpu.make_async_copy(
                tbl_hbm.at[idx_v.at[step]], rows[b], sems[b]
            ).wait()

            @pl.when(step + NBUF < STEPS)
            def _():
                gather(step + NBUF, b)

            reduce_step(b, step)
        return carry

    lax.fori_loop(0, STEPS // NBUF, outer, 0)

    pltpu.sync_copy(out_v, out_hbm.at[pl.ds(obase, BAGS_PER_W)])


@jax.jit
def _embedding_bag(inputs, table):
    idx2d = inputs.reshape(NW * STEPS, IDX_PER_STEP).astype(jnp.int32)
    mesh = plsc.VectorSubcoreMesh(core_axis_name="c", subcore_axis_name="s")
    run = pl.kernel(
        _body,
        out_type=jax.ShapeDtypeStruct((BATCH, EMBED_DIM), jnp.float32),
        mesh=mesh,
        compiler_params=pltpu.CompilerParams(use_tc_tiling_on_sc=False),
        scratch_types=[
            pltpu.VMEM((STEPS, IDX_PER_STEP), jnp.int32),
            pltpu.VMEM((IDX_PER_STEP, EMBED_DIM), jnp.float32),
            pltpu.VMEM((IDX_PER_STEP, EMBED_DIM), jnp.float32),
            pltpu.VMEM((BAGS_PER_W, EMBED_DIM), jnp.float32),
            pltpu.SemaphoreType.DMA,
            pltpu.SemaphoreType.DMA,
        ],
    )
    return run(table, idx2d)


def kernel(inputs, table):
    return _embedding_bag(inputs, table)
